# baseline (speedup 1.0000x reference)
import functools
import math

import jax
import jax.numpy as jnp
from jax import lax
from jax.experimental import pallas as pl
from jax.experimental.pallas import tpu as pltpu
from jax.experimental.pallas import tpu_sc as plsc

CUT1 = 20000
CUT2 = 200000
D0, D1, D2 = 128, 32, 8
ODIM = 128
S0 = math.sqrt(D0)
S1 = math.sqrt(D1)
S2 = math.sqrt(D2)

NSAMP = 4096
SEQ = 50
SEQP = 56
TP = NSAMP * SEQP
NC, NS, L = 2, 16, 16
NW = NC * NS
CHUNK = TP // NW
GB = 128
NJ = CHUNK // GB


_AB = 1024


def _pad_body(i_ref, o_ref):
    pad = jnp.full((_AB, SEQP - SEQ), CUT2, jnp.int32)
    o_ref[...] = jnp.concatenate([i_ref[...], pad], axis=1)


_tc_pad = pl.pallas_call(
    _pad_body,
    out_shape=jax.ShapeDtypeStruct((NSAMP, SEQP), jnp.int32),
    grid=(NSAMP // _AB,),
    in_specs=[pl.BlockSpec((_AB, SEQ), lambda i: (i, 0))],
    out_specs=pl.BlockSpec((_AB, SEQP), lambda i: (i, 0)),
)


_sc_mesh = plsc.VectorSubcoreMesh(core_axis_name="c", subcore_axis_name="s")
_sc_linear = pltpu.CompilerParams(use_tc_tiling_on_sc=False)


@functools.partial(
    pl.kernel,
    out_type=(
        jax.ShapeDtypeStruct((TP, D0), jnp.float32),
        jax.ShapeDtypeStruct((TP, D1), jnp.float32),
        jax.ShapeDtypeStruct((TP, D2), jnp.float32),
    ),
    mesh=_sc_mesh,
    scratch_types=[
        pltpu.VMEM((CHUNK,), jnp.int32),
        pltpu.VMEM((CHUNK,), jnp.int32),
        pltpu.VMEM((CHUNK,), jnp.int32),
        pltpu.VMEM((CHUNK,), jnp.int32),
        pltpu.VMEM((2, GB, D0), jnp.float32),
        pltpu.VMEM((2, GB, D1), jnp.float32),
        pltpu.VMEM((2, GB, D2), jnp.float32),
        pltpu.SemaphoreType.DMA,
        pltpu.SemaphoreType.DMA,
        pltpu.SemaphoreType.DMA,
        pltpu.SemaphoreType.DMA,
        pltpu.SemaphoreType.DMA,
        pltpu.SemaphoreType.DMA,
    ],
    compiler_params=_sc_linear,
)
def _sc_gather(ids_hbm, emb0_hbm, emb1_hbm, emb2_hbm, x0_hbm, x1_hbm, x2_hbm,
               ids_v, idx0_v, idx1_v, idx2_v, g0_v, g1_v, g2_v,
               s0a, s0b, s1a, s1b, s2a, s2b):
    wid = lax.axis_index("s") * NC + lax.axis_index("c")
    base = wid * CHUNK
    pltpu.sync_copy(ids_hbm.at[pl.ds(base, CHUNK)], ids_v)

    @pl.loop(0, CHUNK // 16)
    def _(i):
        o = i * 16
        v = ids_v[pl.ds(o, 16)]
        b0 = v < CUT1
        b1 = (v >= CUT1) & (v < CUT2)
        b2 = v >= CUT2
        idx0_v[pl.ds(o, 16)] = jnp.where(b0, v, 0)
        idx1_v[pl.ds(o, 16)] = jnp.where(b1, v - CUT1, 0)
        idx2_v[pl.ds(o, 16)] = jnp.where(b2, v - CUT2, 0)

    sems0 = (s0a, s0b)
    sems1 = (s1a, s1b)
    sems2 = (s2a, s2b)

    @pl.loop(0, NJ, step=2)
    def _(j):
        descs = []
        for b in range(2):
            o = (j + b) * GB
            descs.append(pltpu.async_copy(
                emb0_hbm.at[idx0_v.at[pl.ds(o, GB)]], g0_v.at[b], sems0[b]))
            descs.append(pltpu.async_copy(
                emb1_hbm.at[idx1_v.at[pl.ds(o, GB)]], g1_v.at[b], sems1[b]))
            descs.append(pltpu.async_copy(
                emb2_hbm.at[idx2_v.at[pl.ds(o, GB)]], g2_v.at[b], sems2[b]))
        for b in range(2):
            o = (j + b) * GB
            descs[3 * b].wait()
            descs[3 * b + 1].wait()
            descs[3 * b + 2].wait()
            pltpu.sync_copy(g0_v.at[b], x0_hbm.at[pl.ds(base + o, GB), :])
            pltpu.sync_copy(g1_v.at[b], x1_hbm.at[pl.ds(base + o, GB), :])
            pltpu.sync_copy(g2_v.at[b], x2_hbm.at[pl.ds(base + o, GB), :])


_BS = 32
_BT = _BS * SEQP


def _proj_body(x0_ref, x1_ref, x2_ref, w0_ref, w1_ref, w2_ref, o_ref):
    a0 = lax.dot_general(
        x0_ref[...], w0_ref[...] * S0, (((1,), (1,)), ((), ())),
        preferred_element_type=jnp.float32)
    a1 = lax.dot_general(
        x1_ref[...], w1_ref[...] * S1, (((1,), (1,)), ((), ())),
        preferred_element_type=jnp.float32)
    a2 = lax.dot_general(
        x2_ref[...], w2_ref[...] * S2, (((1,), (1,)), ((), ())),
        preferred_element_type=jnp.float32)
    a = a0 + a1 + a2
    for s in range(_BS):
        o_ref[s, :, :] = a[s * SEQP:s * SEQP + SEQ, :]


_tc_proj = pl.pallas_call(
    _proj_body,
    out_shape=jax.ShapeDtypeStruct((NSAMP, SEQ, ODIM), jnp.float32),
    grid=(NSAMP // _BS,),
    in_specs=[
        pl.BlockSpec((_BT, D0), lambda i: (i, 0)),
        pl.BlockSpec((_BT, D1), lambda i: (i, 0)),
        pl.BlockSpec((_BT, D2), lambda i: (i, 0)),
        pl.BlockSpec((ODIM, D0), lambda i: (0, 0)),
        pl.BlockSpec((ODIM, D1), lambda i: (0, 0)),
        pl.BlockSpec((ODIM, D2), lambda i: (0, 0)),
    ],
    out_specs=pl.BlockSpec((_BS, SEQ, ODIM), lambda i: (i, 0, 0)),
)


def kernel(input, emb0, emb1, emb2, W0, W1, W2):
    emb1p = jnp.pad(emb1, ((0, (-emb1.shape[0]) % 128), (0, 0)))
    ids = _tc_pad(input).reshape(TP)
    x0, x1, x2 = _sc_gather(ids, emb0, emb1p, emb2)
    return _tc_proj(x0, x1, x2, W0, W1, W2)


# speedup vs baseline: 1.7438x; 1.7438x over previous
import functools
import math

import jax
import jax.numpy as jnp
from jax import lax
from jax.experimental import pallas as pl
from jax.experimental.pallas import tpu as pltpu
from jax.experimental.pallas import tpu_sc as plsc

CUT1 = 20000
CUT2 = 200000
D0, D1, D2 = 128, 32, 8
ODIM = 128
S0 = math.sqrt(D0)
S1 = math.sqrt(D1)
S2 = math.sqrt(D2)

NSAMP = 4096
SEQ = 50
SEQP = 56
TP = NSAMP * SEQP
NC, NS, L = 2, 16, 16
NW = NC * NS
CHUNK = TP // NW
GB = 128
NJ = CHUNK // GB
NB = 7
KLA = 4

BF = jnp.bfloat16


_AB = 1024


def _pad_body(i_ref, o_ref):
    pad = jnp.full((_AB, SEQP - SEQ), CUT2, jnp.int32)
    o_ref[...] = jnp.concatenate([i_ref[...], pad], axis=1)


_tc_pad = pl.pallas_call(
    _pad_body,
    out_shape=jax.ShapeDtypeStruct((NSAMP, SEQP), jnp.int32),
    grid=(NSAMP // _AB,),
    in_specs=[pl.BlockSpec((_AB, SEQ), lambda i: (i, 0))],
    out_specs=pl.BlockSpec((_AB, SEQP), lambda i: (i, 0)),
)


_sc_mesh = plsc.VectorSubcoreMesh(core_axis_name="c", subcore_axis_name="s")
_sc_linear = pltpu.CompilerParams(use_tc_tiling_on_sc=False)


@functools.partial(
    pl.kernel,
    out_type=(
        jax.ShapeDtypeStruct((TP, D0), BF),
        jax.ShapeDtypeStruct((TP, D1), BF),
        jax.ShapeDtypeStruct((TP, D2), jnp.float32),
    ),
    mesh=_sc_mesh,
    scratch_types=[
        pltpu.VMEM((CHUNK,), jnp.int32),
        pltpu.VMEM((CHUNK,), jnp.int32),
        pltpu.VMEM((CHUNK,), jnp.int32),
        pltpu.VMEM((CHUNK,), jnp.int32),
        pltpu.VMEM((NB, GB, D0), BF),
        pltpu.VMEM((NB, GB, D1), BF),
        pltpu.VMEM((NB, GB, D2), jnp.float32),
        pltpu.SemaphoreType.DMA,
        pltpu.SemaphoreType.DMA,
        pltpu.SemaphoreType.DMA,
        pltpu.SemaphoreType.DMA,
        pltpu.SemaphoreType.DMA,
        pltpu.SemaphoreType.DMA,
        pltpu.SemaphoreType.DMA,
    ],
    compiler_params=_sc_linear,
)
def _sc_gather(ids_hbm, emb0_hbm, emb1_hbm, emb2_hbm, x0_hbm, x1_hbm, x2_hbm,
               ids_v, idx0_v, idx1_v, idx2_v, g0_v, g1_v, g2_v,
               sm0, sm1, sm2, sm3, sm4, sm5, sm6):
    wid = lax.axis_index("s") * NC + lax.axis_index("c")
    base = wid * CHUNK
    pltpu.sync_copy(ids_hbm.at[pl.ds(base, CHUNK)], ids_v)

    @pl.loop(0, CHUNK // 16)
    def _(i):
        o = i * 16
        v = ids_v[pl.ds(o, 16)]
        b0 = v < CUT1
        b1 = (v >= CUT1) & (v < CUT2)
        b2 = v >= CUT2
        idx0_v[pl.ds(o, 16)] = jnp.where(b0, v, 0)
        idx1_v[pl.ds(o, 16)] = jnp.where(b1, v - CUT1, 0)
        idx2_v[pl.ds(o, 16)] = jnp.where(b2, v - CUT2, 0)

    embs = (emb0_hbm, emb1_hbm, emb2_hbm)
    xs = (x0_hbm, x1_hbm, x2_hbm)
    idxs = (idx0_v, idx1_v, idx2_v)
    gs = (g0_v, g1_v, g2_v)
    sems = (sm0, sm1, sm2, sm3, sm4, sm5, sm6)

    @pl.loop(0, NJ, step=NB)
    def _(j):
        gd = []
        for b in range(NB):
            o = (j + b) * GB
            for s in range(3):
                gd.append(pltpu.async_copy(
                    embs[s].at[idxs[s].at[pl.ds(o, GB)]], gs[s].at[b],
                    sems[b]))
        wd = []
        for b in range(NB):
            o = (j + b) * GB
            for s in range(3):
                gd[b * 3 + s].wait()
            for s in range(3):
                wd.append(pltpu.async_copy(
                    gs[s].at[b], xs[s].at[pl.ds(base + o, GB), :], sems[b]))
        for d in wd:
            d.wait()


_BS = 32
_BT = _BS * SEQP


def _proj_body(x0_ref, x1_ref, x2_ref, w0_ref, w1_ref, w2_ref, o_ref):
    a0 = lax.dot_general(
        x0_ref[...], (w0_ref[...] * S0).astype(BF), (((1,), (1,)), ((), ())),
        preferred_element_type=jnp.float32)
    a1 = lax.dot_general(
        x1_ref[...], (w1_ref[...] * S1).astype(BF), (((1,), (1,)), ((), ())),
        preferred_element_type=jnp.float32)
    a2 = lax.dot_general(
        x2_ref[...], w2_ref[...] * S2, (((1,), (1,)), ((), ())),
        preferred_element_type=jnp.float32)
    a = a0 + a1 + a2
    for s in range(_BS):
        o_ref[s, :, :] = a[s * SEQP:s * SEQP + SEQ, :]


_tc_proj = pl.pallas_call(
    _proj_body,
    out_shape=jax.ShapeDtypeStruct((NSAMP, SEQ, ODIM), jnp.float32),
    grid=(NSAMP // _BS,),
    in_specs=[
        pl.BlockSpec((_BT, D0), lambda i: (i, 0)),
        pl.BlockSpec((_BT, D1), lambda i: (i, 0)),
        pl.BlockSpec((_BT, D2), lambda i: (i, 0)),
        pl.BlockSpec((ODIM, D0), lambda i: (0, 0)),
        pl.BlockSpec((ODIM, D1), lambda i: (0, 0)),
        pl.BlockSpec((ODIM, D2), lambda i: (0, 0)),
    ],
    out_specs=pl.BlockSpec((_BS, SEQ, ODIM), lambda i: (i, 0, 0)),
)


def kernel(input, emb0, emb1, emb2, W0, W1, W2):
    emb0b = emb0.astype(BF)
    emb1b = jnp.pad(emb1, ((0, (-emb1.shape[0]) % 256), (0, 0))).astype(BF)
    emb2b = emb2
    ids = _tc_pad(input).reshape(TP)
    x0, x1, x2 = _sc_gather(ids, emb0b, emb1b, emb2b)
    return _tc_proj(x0, x1, x2, W0, W1, W2)


# speedup vs baseline: 3.6133x; 2.0721x over previous
import functools
import math

import jax
import jax.numpy as jnp
from jax import lax
from jax.experimental import pallas as pl
from jax.experimental.pallas import tpu as pltpu
from jax.experimental.pallas import tpu_sc as plsc

CUT1 = 20000
CUT2 = 200000
D0, D1, D2 = 128, 32, 8
ODIM = 128
S0 = math.sqrt(D0)
S1 = math.sqrt(D1)
S2 = math.sqrt(D2)

NSAMP = 4096
SEQ = 50
SEQP = 56
TP = NSAMP * SEQP
NC, NS, L = 2, 16, 16
NW = NC * NS
CHUNK = TP // NW
GB = 128
NJ = CHUNK // GB
NB = 7
KLA = 4

BF = jnp.bfloat16


_AB = 1024


def _pad_body(i_ref, o_ref):
    pad = jnp.full((_AB, SEQP - SEQ), CUT2, jnp.int32)
    o_ref[...] = jnp.concatenate([i_ref[...], pad], axis=1)


_tc_pad = pl.pallas_call(
    _pad_body,
    out_shape=jax.ShapeDtypeStruct((NSAMP, SEQP), jnp.int32),
    grid=(NSAMP // _AB,),
    in_specs=[pl.BlockSpec((_AB, SEQ), lambda i: (i, 0))],
    out_specs=pl.BlockSpec((_AB, SEQP), lambda i: (i, 0)),
)


E1ROWS = 180224


def _cast_body(i_ref, o_ref):
    o_ref[...] = i_ref[...].astype(BF)


_tc_cast0 = pl.pallas_call(
    _cast_body,
    out_shape=jax.ShapeDtypeStruct((20000, D0), BF),
    grid=(10,),
    in_specs=[pl.BlockSpec((2000, D0), lambda i: (i, 0))],
    out_specs=pl.BlockSpec((2000, D0), lambda i: (i, 0)),
)

_tc_cast1 = pl.pallas_call(
    _cast_body,
    out_shape=jax.ShapeDtypeStruct((E1ROWS, D1), BF),
    grid=(11,),
    in_specs=[pl.BlockSpec((16384, D1), lambda i: (i, 0))],
    out_specs=pl.BlockSpec((16384, D1), lambda i: (i, 0)),
)


_sc_mesh = plsc.VectorSubcoreMesh(core_axis_name="c", subcore_axis_name="s")
_sc_linear = pltpu.CompilerParams(use_tc_tiling_on_sc=False)


@functools.partial(
    pl.kernel,
    out_type=(
        jax.ShapeDtypeStruct((TP, D0), BF),
        jax.ShapeDtypeStruct((TP, D1), BF),
        jax.ShapeDtypeStruct((TP, D2), jnp.float32),
    ),
    mesh=_sc_mesh,
    scratch_types=[
        pltpu.VMEM((CHUNK,), jnp.int32),
        pltpu.VMEM((CHUNK,), jnp.int32),
        pltpu.VMEM((CHUNK,), jnp.int32),
        pltpu.VMEM((CHUNK,), jnp.int32),
        pltpu.VMEM((NB, GB, D0), BF),
        pltpu.VMEM((GB,), jnp.int32),
        pltpu.VMEM((NB, GB, D1), BF),
        pltpu.VMEM((NB, GB, D2), jnp.float32),
        pltpu.SemaphoreType.DMA,
        pltpu.SemaphoreType.DMA,
        pltpu.SemaphoreType.DMA,
        pltpu.SemaphoreType.DMA,
        pltpu.SemaphoreType.DMA,
        pltpu.SemaphoreType.DMA,
        pltpu.SemaphoreType.DMA,
    ],
    compiler_params=_sc_linear,
)
def _sc_gather(ids_hbm, emb0_hbm, emb1_hbm, emb2_hbm, x0_hbm, x1_hbm, x2_hbm,
               ids_v, idx0_v, idx1_v, idx2_v, g0_v, zidx_v, g1_v, g2_v,
               sm0, sm1, sm2, sm3, sm4, sm5, sm6):
    wid = lax.axis_index("s") * NC + lax.axis_index("c")
    base = wid * CHUNK
    pltpu.sync_copy(ids_hbm.at[pl.ds(base, CHUNK)], ids_v)

    @pl.loop(0, CHUNK // 16)
    def _(i):
        o = i * 16
        v = ids_v[pl.ds(o, 16)]
        b0 = v < CUT1
        b1 = (v >= CUT1) & (v < CUT2)
        b2 = v >= CUT2
        idx0_v[pl.ds(o, 16)] = jnp.where(b0, v, 0)
        idx1_v[pl.ds(o, 16)] = jnp.where(b1, v - CUT1, 0)
        idx2_v[pl.ds(o, 16)] = jnp.where(b2, v - CUT2, 0)

    embs = (emb0_hbm, emb1_hbm, emb2_hbm)
    xs = (x0_hbm, x1_hbm, x2_hbm)
    idxs = (idx0_v, idx1_v, idx2_v)
    gs = (g0_v, g1_v, g2_v)
    sems = (sm0, sm1, sm2, sm3, sm4, sm5, sm6)

    zero16i = jnp.zeros((16,), jnp.int32)
    for c in range(GB // 16):
        zidx_v[pl.ds(c * 16, 16)] = zero16i
    zd = [pltpu.async_copy(emb0_hbm.at[zidx_v], g0_v.at[b], sems[b])
          for b in range(NB)]
    for d in zd:
        d.wait()

    @pl.loop(0, NJ, step=NB)
    def _(j):
        gd = []
        for b in range(NB):
            o = (j + b) * GB
            for s in range(1, 3):
                gd.append(pltpu.async_copy(
                    embs[s].at[idxs[s].at[pl.ds(o, GB)]], gs[s].at[b],
                    sems[b]))

        @pl.loop(0, NB)
        def _(bb):
            @pl.loop(0, GB // 16)
            def _(g):
                v0 = idx0_v[pl.ds((j + bb) * GB + g * 16, 16)]
                for lane in range(16):
                    iv = v0[lane]

                    @pl.when(iv > 0)
                    def _():
                        pltpu.sync_copy(
                            emb0_hbm.at[iv],
                            g0_v.at[bb, g * 16 + lane])

        wd = []
        for b in range(NB):
            o = (j + b) * GB
            for s in range(1, 3):
                gd[b * 2 + (s - 1)].wait()
            for s in range(3):
                wd.append(pltpu.async_copy(
                    gs[s].at[b], xs[s].at[pl.ds(base + o, GB), :], sems[b]))
        for d in wd:
            d.wait()

        @pl.loop(0, NB)
        def _(bb):
            @pl.loop(0, GB // 16)
            def _(g):
                v0 = idx0_v[pl.ds((j + bb) * GB + g * 16, 16)]
                for lane in range(16):
                    iv = v0[lane]

                    @pl.when(iv > 0)
                    def _():
                        pltpu.sync_copy(
                            emb0_hbm.at[0],
                            g0_v.at[bb, g * 16 + lane])


_BS = 32
_BT = _BS * SEQP


def _proj_body(x0_ref, x1_ref, x2_ref, w0_ref, w1_ref, w2_ref, o_ref):
    a0 = lax.dot_general(
        x0_ref[...], (w0_ref[...] * S0).astype(BF), (((1,), (1,)), ((), ())),
        preferred_element_type=jnp.float32)
    a1 = lax.dot_general(
        x1_ref[...], (w1_ref[...] * S1).astype(BF), (((1,), (1,)), ((), ())),
        preferred_element_type=jnp.float32)
    a2 = lax.dot_general(
        x2_ref[...], w2_ref[...] * S2, (((1,), (1,)), ((), ())),
        preferred_element_type=jnp.float32)
    a = a0 + a1 + a2
    for s in range(_BS):
        o_ref[s, :, :] = a[s * SEQP:s * SEQP + SEQ, :]


_tc_proj = pl.pallas_call(
    _proj_body,
    out_shape=jax.ShapeDtypeStruct((NSAMP, SEQ, ODIM), jnp.float32),
    grid=(NSAMP // _BS,),
    in_specs=[
        pl.BlockSpec((_BT, D0), lambda i: (i, 0)),
        pl.BlockSpec((_BT, D1), lambda i: (i, 0)),
        pl.BlockSpec((_BT, D2), lambda i: (i, 0)),
        pl.BlockSpec((ODIM, D0), lambda i: (0, 0)),
        pl.BlockSpec((ODIM, D1), lambda i: (0, 0)),
        pl.BlockSpec((ODIM, D2), lambda i: (0, 0)),
    ],
    out_specs=pl.BlockSpec((_BS, SEQ, ODIM), lambda i: (i, 0, 0)),
)


def kernel(input, emb0, emb1, emb2, W0, W1, W2):
    emb0b = _tc_cast0(emb0)
    emb1b = _tc_cast1(emb1)
    emb2b = emb2
    ids = _tc_pad(input).reshape(TP)
    x0, x1, x2 = _sc_gather(ids, emb0b, emb1b, emb2b)
    return _tc_proj(x0, x1, x2, W0, W1, W2)
